# baseline (speedup 1.0000x reference)
import functools

import jax
import jax.numpy as jnp
from jax import lax
from jax.experimental import pallas as pl
from jax.experimental.pallas import tpu as pltpu
from jax.experimental.pallas import tpu_sc as plsc

_B = 16384
_D = 32
_FF = 13
_NF = 26
_NR = _NF + 1
_NCOLS = _FF + _NF
_VOCAB = 100000
_V = _VOCAB * _NF

_NC = 2
_NS = 16
_NW = _NC * _NS
_BW = _B // _NW
_CB = 128
_NSUB = _BW // _CB

_PR = _V // 4
_RC = 80
_NCHUNK = _PR // _RC


def _precompute(features, W, b):
    BS = 2048

    def body(f_ref, w_ref, b_ref, d_ref, i_ref):
        x = f_ref[:, :_FF].astype(jnp.float32)
        d_ref[...] = (
            lax.dot_general(
                x, w_ref[...], (((1,), (1,)), ((), ())),
                preferred_element_type=jnp.float32,
            )
            + b_ref[...]
        )
        f26 = lax.broadcasted_iota(jnp.int32, (BS, _NF), 1)
        i_ref[...] = f_ref[:, _FF:] + f26 * _VOCAB

    return pl.pallas_call(
        body,
        grid=(_B // BS,),
        in_specs=[
            pl.BlockSpec((BS, _NCOLS), lambda i: (i, 0)),
            pl.BlockSpec((_D, _FF), lambda i: (0, 0)),
            pl.BlockSpec((1, _D), lambda i: (0, 0)),
        ],
        out_specs=[
            pl.BlockSpec((BS, _D), lambda i: (i, 0)),
            pl.BlockSpec((BS, _NF), lambda i: (i, 0)),
        ],
        out_shape=[
            jax.ShapeDtypeStruct((_B, _D), jnp.float32),
            jax.ShapeDtypeStruct((_B, _NF), jnp.int32),
        ],
    )(features, W, b.reshape(1, _D))


def _repack_table(table):
    mesh = plsc.VectorSubcoreMesh(core_axis_name="c", subcore_axis_name="s")

    @functools.partial(
        pl.kernel,
        mesh=mesh,
        out_type=jax.ShapeDtypeStruct((_PR, 128), jnp.float32),
        scratch_types=[
            pltpu.VMEM((_RC * 4, _D), jnp.float32),
            pltpu.VMEM((_RC, 128), jnp.float32),
        ],
    )
    def k(table_hbm, out_hbm, vin, vout):
        wid = lax.axis_index("s") * _NC + lax.axis_index("c")
        nw = (_NCHUNK - wid + _NW - 1) // _NW

        def chunk_body(kk, carry):
            ch = wid + kk * _NW
            pltpu.sync_copy(table_hbm.at[pl.ds(ch * _RC * 4, _RC * 4)], vin)

            def pack_row(j, c2):
                for q in range(4):
                    src = vin.at[j * 4 + q]
                    v0 = src[pl.ds(0, 16)]
                    v1 = src[pl.ds(16, 16)]
                    vout[j, pl.ds(q * _D, 16)] = v0
                    vout[j, pl.ds(q * _D + 16, 16)] = v1
                return c2

            lax.fori_loop(0, _RC, pack_row, 0)
            pltpu.sync_copy(vout, out_hbm.at[pl.ds(ch * _RC, _RC)])
            return carry

        lax.fori_loop(0, nw, chunk_body, 0)

    return k(table)


def _sc_assemble(idx, dense, table_lin):
    mesh = plsc.VectorSubcoreMesh(core_axis_name="c", subcore_axis_name="s")

    @functools.partial(
        pl.kernel,
        mesh=mesh,
        compiler_params=pltpu.CompilerParams(use_tc_tiling_on_sc=False),
        out_type=jax.ShapeDtypeStruct((_B * _NR, _D), jnp.float32),
        scratch_types=[
            pltpu.VMEM((_CB, _NF), jnp.int32),
            pltpu.VMEM((_CB * _NR, _D), jnp.float32),
            pltpu.SemaphoreType.DMA,
            pltpu.SemaphoreType.DMA,
        ],
    )
    def k(idx_hbm, dense_hbm, table_hbm, out_hbm, idx_v, gbuf, gsem, dsem):
        wid = lax.axis_index("s") * _NC + lax.axis_index("c")
        base = wid * _BW

        for sub in range(_NSUB):
            b0 = base + sub * _CB
            pltpu.sync_copy(idx_hbm.at[pl.ds(b0, _CB)], idx_v)

            def fire(bb, carry):
                pltpu.async_copy(
                    table_hbm.at[idx_v.at[bb]],
                    gbuf.at[pl.ds(bb * _NR + 1, _NF)],
                    gsem,
                )
                pltpu.async_copy(
                    dense_hbm.at[pl.ds(b0 + bb, 1)],
                    gbuf.at[pl.ds(bb * _NR, 1)],
                    dsem,
                )
                return carry

            lax.fori_loop(0, _CB, fire, 0)

            def drain(bb, carry):
                pltpu.make_async_copy(
                    table_hbm.at[idx_v.at[0]],
                    gbuf.at[pl.ds(1, _NF)],
                    gsem,
                ).wait()
                pltpu.make_async_copy(
                    dense_hbm.at[pl.ds(b0, 1)],
                    gbuf.at[pl.ds(0, 1)],
                    dsem,
                ).wait()
                return carry

            lax.fori_loop(0, _CB, drain, 0)

            pltpu.sync_copy(gbuf, out_hbm.at[pl.ds(b0 * _NR, _CB * _NR)])

    return k(idx, dense, table_lin)


def kernel(original_features, table, W, b):
    dense, idx = _precompute(original_features, W, b)
    table_lin = _repack_table(table).reshape(_V, _D)
    out2d = _sc_assemble(idx, dense, table_lin)
    return out2d.reshape(_B, _NR, _D)


# speedup vs baseline: 1.1757x; 1.1757x over previous
import functools

import jax
import jax.numpy as jnp
from jax import lax
from jax.experimental import pallas as pl
from jax.experimental.pallas import tpu as pltpu
from jax.experimental.pallas import tpu_sc as plsc

_B = 16384
_D = 32
_FF = 13
_NF = 26
_NR = _NF + 1
_NCOLS = _FF + _NF
_VOCAB = 100000

_NC = 2
_NS = 16
_NW = _NC * _NS
_BW = _B // _NW
_CB = 128
_NSUB = _BW // _CB
_CR = _CB * _NR
_IR = _CR // 128


def _precompute(features, W, b):
    BS = 2048

    def body(f_ref, w_ref, b_ref, d_ref, i_ref):
        x = f_ref[:, :_FF].astype(jnp.float32)
        d_ref[...] = (
            lax.dot_general(
                x, w_ref[...], (((1,), (1,)), ((), ())),
                preferred_element_type=jnp.float32,
            )
            + b_ref[...]
        )
        f26 = lax.broadcasted_iota(jnp.int32, (BS, _NF), 1)
        tok = f_ref[:, _FF:] + f26 * _VOCAB
        i_ref[...] = jnp.concatenate(
            [jnp.zeros((BS, 1), jnp.int32), tok], axis=1
        )

    return pl.pallas_call(
        body,
        grid=(_B // BS,),
        in_specs=[
            pl.BlockSpec((BS, _NCOLS), lambda i: (i, 0)),
            pl.BlockSpec((_D, _FF), lambda i: (0, 0)),
            pl.BlockSpec((1, _D), lambda i: (0, 0)),
        ],
        out_specs=[
            pl.BlockSpec((BS, _D), lambda i: (i, 0)),
            pl.BlockSpec((BS, _NR), lambda i: (i, 0)),
        ],
        out_shape=[
            jax.ShapeDtypeStruct((_B, _D), jnp.float32),
            jax.ShapeDtypeStruct((_B, _NR), jnp.int32),
        ],
    )(features, W, b.reshape(1, _D))


def _sc_assemble(idx_rows, dense, table, dloc):
    mesh = plsc.VectorSubcoreMesh(core_axis_name="c", subcore_axis_name="s")

    @functools.partial(
        pl.kernel,
        mesh=mesh,
        compiler_params=pltpu.CompilerParams(use_tc_tiling_on_sc=False),
        out_type=jax.ShapeDtypeStruct((_B * _NR, _D), jnp.float32),
        scratch_types=[
            pltpu.VMEM((_IR, 128), jnp.int32),
            pltpu.VMEM((_CR, _D), jnp.float32),
            pltpu.VMEM((_CB, _D), jnp.float32),
            pltpu.VMEM((128,), jnp.int32),
            pltpu.SemaphoreType.DMA,
            pltpu.SemaphoreType.DMA,
        ],
    )
    def k(idx_hbm, dense_hbm, table_hbm, dloc_hbm, out_hbm,
          idx_v, gbuf, dvals, dloc_v, gsem, ssem):
        wid = lax.axis_index("s") * _NC + lax.axis_index("c")
        base = wid * _BW

        for sub in range(_NSUB):
            b0 = base + sub * _CB
            r0 = b0 * _NR // 128
            pltpu.sync_copy(idx_hbm.at[pl.ds(r0, _IR)], idx_v)
            pltpu.sync_copy(dloc_hbm.at[b0 // _CB], dloc_v)

            for r in range(_IR):
                pltpu.async_copy(
                    table_hbm.at[idx_v.at[r]],
                    gbuf.at[pl.ds(r * 128, 128)],
                    gsem,
                )
            for r in range(_IR):
                pltpu.make_async_copy(
                    table_hbm.at[idx_v.at[0]],
                    gbuf.at[pl.ds(0, 128)],
                    gsem,
                ).wait()

            pltpu.sync_copy(dense_hbm.at[pl.ds(b0, _CB)], dvals)
            pltpu.sync_copy(gbuf, out_hbm.at[pl.ds(b0 * _NR, _CR)])
            pltpu.async_copy(dvals, out_hbm.at[dloc_v], ssem).wait()

    return k(idx_rows, dense, table, dloc)


def kernel(original_features, table, W, b):
    dense, idx_ext = _precompute(original_features, W, b)
    idx_rows = idx_ext.reshape(_B * _NR // 128, 128)
    dloc = (jnp.arange(_B, dtype=jnp.int32) * _NR).reshape(_B // _CB, _CB)
    out2d = _sc_assemble(idx_rows, dense, table, dloc)
    return out2d.reshape(_B, _NR, _D)


# speedup vs baseline: 1.1783x; 1.0022x over previous
import functools

import jax
import jax.numpy as jnp
from jax import lax
from jax.experimental import pallas as pl
from jax.experimental.pallas import tpu as pltpu
from jax.experimental.pallas import tpu_sc as plsc

_B = 16384
_D = 32
_FF = 13
_NF = 26
_NR = _NF + 1
_NCOLS = _FF + _NF
_VOCAB = 100000

_NC = 2
_NS = 16
_NW = _NC * _NS
_BW = _B // _NW
_CB = 128
_NSUB = _BW // _CB
_CR = _CB * _NR
_IR = _CR // 128


def _precompute(features, W, b):
    BS = 2048

    def body(f_ref, w_ref, b_ref, d_ref, i_ref):
        x = f_ref[:, :_FF].astype(jnp.float32)
        d_ref[...] = (
            lax.dot_general(
                x, w_ref[...], (((1,), (1,)), ((), ())),
                preferred_element_type=jnp.float32,
            )
            + b_ref[...]
        )
        f26 = lax.broadcasted_iota(jnp.int32, (BS, _NF), 1)
        tok = f_ref[:, _FF:] + f26 * _VOCAB
        i_ref[...] = jnp.concatenate(
            [jnp.zeros((BS, 1), jnp.int32), tok], axis=1
        )

    return pl.pallas_call(
        body,
        grid=(_B // BS,),
        in_specs=[
            pl.BlockSpec((BS, _NCOLS), lambda i: (i, 0)),
            pl.BlockSpec((_D, _FF), lambda i: (0, 0)),
            pl.BlockSpec((1, _D), lambda i: (0, 0)),
        ],
        out_specs=[
            pl.BlockSpec((BS, _D), lambda i: (i, 0)),
            pl.BlockSpec((BS, _NR), lambda i: (i, 0)),
        ],
        out_shape=[
            jax.ShapeDtypeStruct((_B, _D), jnp.float32),
            jax.ShapeDtypeStruct((_B, _NR), jnp.int32),
        ],
    )(features, W, b.reshape(1, _D))


def _sc_assemble(idx_rows, dense, table, dloc):
    mesh = plsc.VectorSubcoreMesh(core_axis_name="c", subcore_axis_name="s")

    @functools.partial(
        pl.kernel,
        mesh=mesh,
        compiler_params=pltpu.CompilerParams(use_tc_tiling_on_sc=False),
        out_type=jax.ShapeDtypeStruct((_B * _NR, _D), jnp.float32),
        scratch_types=[
            pltpu.VMEM((_CR,), jnp.int32),
            pltpu.VMEM((_CR, _D), jnp.float32),
            pltpu.VMEM((_CB, _D), jnp.float32),
            pltpu.VMEM((128,), jnp.int32),
            pltpu.SemaphoreType.DMA,
            pltpu.SemaphoreType.DMA,
        ],
    )
    def k(idx_hbm, dense_hbm, table_hbm, dloc_hbm, out_hbm,
          idx_v, gbuf, dvals, dloc_v, gsem, ssem):
        wid = lax.axis_index("s") * _NC + lax.axis_index("c")
        base = wid * _BW

        for sub in range(_NSUB):
            b0 = base + sub * _CB
            pltpu.sync_copy(idx_hbm.at[pl.ds(b0 * _NR, _CR)], idx_v)
            pltpu.sync_copy(dloc_hbm.at[b0 // _CB], dloc_v)

            pltpu.async_copy(table_hbm.at[idx_v], gbuf, gsem).wait()

            pltpu.sync_copy(dense_hbm.at[pl.ds(b0, _CB)], dvals)
            pltpu.sync_copy(gbuf, out_hbm.at[pl.ds(b0 * _NR, _CR)])
            pltpu.async_copy(dvals, out_hbm.at[dloc_v], ssem).wait()

    return k(idx_rows, dense, table, dloc)


def kernel(original_features, table, W, b):
    dense, idx_ext = _precompute(original_features, W, b)
    idx_rows = idx_ext.reshape(_B * _NR)
    dloc = (jnp.arange(_B, dtype=jnp.int32) * _NR).reshape(_B // _CB, _CB)
    out2d = _sc_assemble(idx_rows, dense, table, dloc)
    return out2d.reshape(_B, _NR, _D)
